# baseline (speedup 1.0000x reference)
import functools

import jax
import jax.numpy as jnp
from jax import lax
from jax.experimental import pallas as pl
from jax.experimental.pallas import tpu as pltpu
from jax.experimental.pallas import tpu_sc as plsc

F32 = jnp.float32

NC = 2
NS = 16
CH = 80
G = 5


def _make_segsum(n, e, w, compute_deg):
    nw = NC * NS
    epw = e // nw
    rows_pt = (n // NS) // 8 * 8
    tail = n - NS * rows_pt
    iters = epw // (CH * G)
    assert epw % (CH * G) == 0 and tail % 8 == 0

    mesh = plsc.VectorSubcoreMesh(core_axis_name="c", subcore_axis_name="s")

    out_type = [jax.ShapeDtypeStruct((NC, n, w), F32)]
    if compute_deg:
        out_type.append(jax.ShapeDtypeStruct((NC, n), F32))

    scratch = [
        pltpu.VMEM((epw // CH, CH), jnp.int32),
        pltpu.VMEM((epw // CH, CH), jnp.int32),
        pltpu.VMEM((G, CH, w), F32),
        pltpu.VMEM_SHARED((n, w), F32),
        pltpu.SemaphoreType.DMA,
    ]
    if compute_deg:
        scratch += [
            pltpu.VMEM((CH,), F32),
            pltpu.VMEM_SHARED((n,), F32),
        ]

    @functools.partial(
        pl.kernel, mesh=mesh, out_type=out_type, scratch_types=scratch,
        name="sc_segsum_deg" if compute_deg else "sc_segsum",
        compiler_params=pltpu.CompilerParams(use_tc_tiling_on_sc=False),
    )
    def k(*refs):
        if compute_deg:
            (a_hbm, src_hbm, dst_hbm, z2_hbm, z1_hbm, out_hbm, deg_hbm,
             src_v, dst_v, rows_v, acc_sh, sem, ones_v, deg_sh) = refs
        else:
            (a_hbm, src_hbm, dst_hbm, z2_hbm, out_hbm,
             src_v, dst_v, rows_v, acc_sh, sem) = refs
        c = lax.axis_index("c")
        s = lax.axis_index("s")

        base_r = pl.multiple_of(s * rows_pt, 8)
        pltpu.sync_copy(z2_hbm.at[pl.ds(base_r, rows_pt)],
                        acc_sh.at[pl.ds(base_r, rows_pt)])
        if tail:
            @pl.when(s == NS - 1)
            def _():
                pltpu.sync_copy(z2_hbm.at[pl.ds(NS * rows_pt, tail)],
                                acc_sh.at[pl.ds(NS * rows_pt, tail)])
        if compute_deg:
            @pl.when(s == 0)
            def _():
                pltpu.sync_copy(z1_hbm, deg_sh)
            for i in range(CH // 16):
                ones_v[pl.ds(i * 16, 16)] = jnp.ones((16,), F32)

        wid = c * NS + s
        pltpu.sync_copy(src_hbm.at[wid], src_v)
        pltpu.sync_copy(dst_hbm.at[wid], dst_v)
        plsc.subcore_barrier()

        def step(i, carry):
            cps = [pltpu.async_copy(a_hbm.at[src_v.at[i * G + j]],
                                    rows_v.at[j], sem)
                   for j in range(G)]
            for cp in cps:
                cp.wait()
            for j in range(G):
                pltpu.sync_copy(rows_v.at[j], acc_sh.at[dst_v.at[i * G + j]],
                                add=True)
                if compute_deg:
                    pltpu.sync_copy(ones_v, deg_sh.at[dst_v.at[i * G + j]],
                                    add=True)
            return carry

        lax.fori_loop(0, iters, step, 0)
        plsc.subcore_barrier()

        pltpu.sync_copy(acc_sh.at[pl.ds(base_r, rows_pt)],
                        out_hbm.at[c, pl.ds(base_r, rows_pt)])
        if tail:
            @pl.when(s == NS - 1)
            def _():
                pltpu.sync_copy(acc_sh.at[pl.ds(NS * rows_pt, tail)],
                                out_hbm.at[c, pl.ds(NS * rows_pt, tail)])
        if compute_deg:
            @pl.when(s == 0)
            def _():
                pltpu.sync_copy(deg_sh, deg_hbm.at[c])

    return k


def _dotd(a, b):
    return jnp.dot(a, b, preferred_element_type=F32)


def _dot_hi(a, b):
    return jnp.dot(a, b, preferred_element_type=F32,
                   precision=lax.Precision.HIGHEST)


def _bn(pre, g, b):
    m = jnp.mean(pre, axis=0, keepdims=True)
    d = pre - m
    v = jnp.mean(d * d, axis=0, keepdims=True)
    return (pre - m) / jnp.sqrt(v + 1e-5) * g + b


def _pre0_body(parta_ref, partb_ref, degt_ref, x_ref, wl_ref, wr_ref, ws_ref,
               bl_ref, pre_ref, sk_ref):
    deg = jnp.maximum(degt_ref[:, 0:1] + degt_ref[:, 1:2], 1.0)
    mean = jnp.concatenate(
        [(parta_ref[0] + parta_ref[1]) / deg,
         (partb_ref[0] + partb_ref[1]) / deg], axis=1)
    xv = x_ref[...]
    pre_ref[...] = _dotd(mean, wl_ref[...]) + bl_ref[...] + _dotd(xv, wr_ref[...])
    sk_ref[...] = _dotd(xv, ws_ref[...])


def _pre_body(part_ref, degt_ref, h_ref, wl_ref, wr_ref, bl_ref, pre_ref):
    s = part_ref[0] + part_ref[1]
    deg = degt_ref[:, 0:1] + degt_ref[:, 1:2]
    mean = s / jnp.maximum(deg, 1.0)
    pre_ref[...] = (_dotd(mean, wl_ref[...]) + bl_ref[...]
                    + _dotd(h_ref[...], wr_ref[...]))


def _bnres_relu_body(pre_ref, sk_ref, g_ref, b_ref, h_ref):
    h_ref[...] = jnp.maximum(_bn(pre_ref[...], g_ref[...], b_ref[...]),
                             0.0) + sk_ref[...]


def _bnres_body(pre_ref, sk_ref, g_ref, b_ref, h_ref):
    h_ref[...] = _bn(pre_ref[...], g_ref[...], b_ref[...]) + sk_ref[...]


def _make_pool_mlp(n, h, bsz, kk, cn=1000, sb=64):
    nchunk = n // cn
    nsb = bsz // sb
    assert n % cn == 0 and bsz % sb == 0

    def body(h3_ref, bt_ref, key_ref, w1_ref, b1_ref, w2_ref, b2_ref,
             out_ref, dense_scr, taken_scr, m_scr, sel_scr, acc_scr, act_scr):
        h3 = h3_ref[...]
        neg = jnp.float32(-jnp.inf)
        taken_scr[...] = jnp.zeros((nchunk, cn), F32)

        for c in range(nchunk):
            blk = bt_ref[c:c + 1, :] // sb
            for b in range(nsb):
                act_scr[c, b] = jnp.max(jnp.where(blk == b, 1, 0))

        iot_seg = lax.broadcasted_iota(jnp.int32, (sb, cn), 0)
        iot_col = lax.broadcasted_iota(jnp.int32, (sb, cn), 1)

        def pass_k(k, carry):
            m_scr[...] = jnp.full((bsz, 1), neg, F32)
            sel_scr[...] = jnp.full((bsz, 1), n, jnp.int32)
            acc_scr[...] = jnp.zeros((bsz, h), F32)
            for c in range(nchunk):
                for b in range(nsb):
                    @pl.when(act_scr[c, b] > 0)
                    def _():
                        free = taken_scr[c:c + 1, :] == 0.0
                        maskc = ((bt_ref[c:c + 1, :] - b * sb) == iot_seg) & free
                        mloc = jnp.max(
                            jnp.where(maskc, key_ref[c:c + 1, :], neg),
                            axis=1, keepdims=True)
                        m_scr[b * sb:(b + 1) * sb, :] = jnp.maximum(
                            m_scr[b * sb:(b + 1) * sb, :], mloc)
            for c in range(nchunk):
                for b in range(nsb):
                    @pl.when(act_scr[c, b] > 0)
                    def _():
                        free = taken_scr[c:c + 1, :] == 0.0
                        maskc = ((bt_ref[c:c + 1, :] - b * sb) == iot_seg) & free
                        eq = maskc & (key_ref[c:c + 1, :]
                                      == m_scr[b * sb:(b + 1) * sb, :])
                        sloc = jnp.min(jnp.where(eq, iot_col, n), axis=1,
                                       keepdims=True) + c * cn
                        sel_scr[b * sb:(b + 1) * sb, :] = jnp.minimum(
                            sel_scr[b * sb:(b + 1) * sb, :], sloc)
            for c in range(nchunk):
                for b in range(nsb):
                    @pl.when(act_scr[c, b] > 0)
                    def _():
                        selc = sel_scr[b * sb:(b + 1) * sb, :] - c * cn
                        oh = (iot_col == selc).astype(F32)
                        acc_scr[b * sb:(b + 1) * sb, :] += _dot_hi(
                            oh, h3[c * cn:(c + 1) * cn])
                        hit = jnp.max(oh, axis=0, keepdims=True)
                        taken_scr[c:c + 1, :] = jnp.maximum(
                            taken_scr[c:c + 1, :], hit)
            dense_scr[:, pl.ds(k, 1), :] = jnp.reshape(acc_scr[...],
                                                       (bsz, 1, h))
            return carry

        lax.fori_loop(0, kk, pass_k, 0)
        pooled = jnp.reshape(dense_scr[...], (bsz, kk * h))
        hid = jnp.maximum(_dotd(pooled, w1_ref[...]) + b1_ref[...], 0.0)
        out_ref[...] = _dotd(hid, w2_ref[...]) + b2_ref[...]

    return body


def kernel(x, edge_index, batch, params):
    n, d = x.shape
    e = edge_index.shape[1]
    hdim = params["conv0_Wl"].shape[1]
    kk = params["mlp_W1"].shape[0] // hdim
    bsz = 256

    nw = NC * NS
    src2 = edge_index[0].reshape(nw, e // nw // CH, CH)
    dst2 = edge_index[1].reshape(nw, e // nw // CH, CH)
    zd = jnp.zeros((n, d), F32)
    zh = jnp.zeros((n, hdim), F32)
    z1 = jnp.zeros((n,), F32)
    cn = 1000
    bt2 = batch.reshape(n // cn, cn)

    def row(v):
        return v.reshape(1, -1)

    rb = 2000

    def wspec(*shape):
        return pl.BlockSpec(shape, lambda i: tuple(0 for _ in shape))

    dh = d // 2
    zdh = jnp.zeros((n, dh), F32)
    seg0 = _make_segsum(n, e, dh, True)
    part0a, degp = seg0(x[:, :dh], src2, dst2, zdh, z1)
    part0b = _make_segsum(n, e, dh, False)(x[:, dh:], src2, dst2, zdh)[0]
    degt = degp.T

    pre0, sk = pl.pallas_call(
        _pre0_body,
        grid=(n // rb,),
        in_specs=[
            pl.BlockSpec((NC, rb, dh), lambda i: (0, i, 0)),
            pl.BlockSpec((NC, rb, dh), lambda i: (0, i, 0)),
            pl.BlockSpec((rb, NC), lambda i: (i, 0)),
            pl.BlockSpec((rb, d), lambda i: (i, 0)),
            wspec(d, hdim), wspec(d, hdim), wspec(d, hdim), wspec(1, hdim),
        ],
        out_specs=[pl.BlockSpec((rb, hdim), lambda i: (i, 0))] * 2,
        out_shape=[jax.ShapeDtypeStruct((n, hdim), F32)] * 2,
    )(part0a, part0b, degt, x, params["conv0_Wl"], params["conv0_Wr"],
      params["skip_W"], row(params["conv0_bl"]))

    h1 = pl.pallas_call(
        _bnres_relu_body,
        out_shape=jax.ShapeDtypeStruct((n, hdim), F32),
    )(pre0, sk, row(params["bn0_g"]), row(params["bn0_b"]))

    seg = _make_segsum(n, e, hdim, False)

    def conv_layer(hv, l, relu):
        part = seg(hv, src2, dst2, zh)[0]
        pre = pl.pallas_call(
            _pre_body,
            grid=(n // rb,),
            in_specs=[
                pl.BlockSpec((NC, rb, hdim), lambda i: (0, i, 0)),
                pl.BlockSpec((rb, NC), lambda i: (i, 0)),
                pl.BlockSpec((rb, hdim), lambda i: (i, 0)),
                wspec(hdim, hdim), wspec(hdim, hdim), wspec(1, hdim),
            ],
            out_specs=pl.BlockSpec((rb, hdim), lambda i: (i, 0)),
            out_shape=jax.ShapeDtypeStruct((n, hdim), F32),
        )(part, degt, hv, params[f"conv{l}_Wl"], params[f"conv{l}_Wr"],
          row(params[f"conv{l}_bl"]))
        return pl.pallas_call(
            _bnres_relu_body if relu else _bnres_body,
            out_shape=jax.ShapeDtypeStruct((n, hdim), F32),
        )(pre, hv, row(params[f"bn{l}_g"]), row(params[f"bn{l}_b"]))

    h2 = conv_layer(h1, 1, True)
    h3 = conv_layer(h2, 2, False)

    key2 = h3[:, hdim - 1].reshape(n // cn, cn)
    sb = 32
    out = pl.pallas_call(
        _make_pool_mlp(n, hdim, bsz, kk, cn, sb),
        out_shape=jax.ShapeDtypeStruct((bsz, 1), F32),
        scratch_shapes=[pltpu.VMEM((bsz, kk, hdim), F32),
                        pltpu.VMEM((n // cn, cn), F32),
                        pltpu.VMEM((bsz, 1), F32),
                        pltpu.VMEM((bsz, 1), jnp.int32),
                        pltpu.VMEM((bsz, hdim), F32),
                        pltpu.SMEM((n // cn, bsz // sb), jnp.int32)],
    )(h3, bt2, key2, params["mlp_W1"], row(params["mlp_b1"]),
      params["mlp_W2"], row(params["mlp_b2"]))

    return out[:, 0]


# speedup vs baseline: 1.5341x; 1.5341x over previous
import functools

import jax
import jax.numpy as jnp
from jax import lax
from jax.experimental import pallas as pl
from jax.experimental.pallas import tpu as pltpu
from jax.experimental.pallas import tpu_sc as plsc

F32 = jnp.float32

NC = 2
NS = 16
CH = 80
G = 5


def _make_segsum(n, e, w, compute_deg):
    nw = NC * NS
    epw = e // nw
    rows_pt = (n // NS) // 8 * 8
    tail = n - NS * rows_pt
    iters = epw // (CH * G)
    assert epw % (CH * G) == 0 and tail % 8 == 0

    mesh = plsc.VectorSubcoreMesh(core_axis_name="c", subcore_axis_name="s")

    out_type = [jax.ShapeDtypeStruct((NC, n, w), F32)]
    if compute_deg:
        out_type.append(jax.ShapeDtypeStruct((NC, n), F32))

    scratch = [
        pltpu.VMEM((epw // CH, CH), jnp.int32),
        pltpu.VMEM((epw // CH, CH), jnp.int32),
        pltpu.VMEM((G, CH, w), F32),
        pltpu.VMEM_SHARED((n, w), F32),
        pltpu.SemaphoreType.DMA,
    ]
    if compute_deg:
        scratch += [
            pltpu.VMEM((CH,), F32),
            pltpu.VMEM_SHARED((n,), F32),
        ]

    @functools.partial(
        pl.kernel, mesh=mesh, out_type=out_type, scratch_types=scratch,
        name="sc_segsum_deg" if compute_deg else "sc_segsum",
        compiler_params=pltpu.CompilerParams(use_tc_tiling_on_sc=False),
    )
    def k(*refs):
        if compute_deg:
            (a_hbm, src_hbm, dst_hbm, z2_hbm, z1_hbm, out_hbm, deg_hbm,
             src_v, dst_v, rows_v, acc_sh, sem, ones_v, deg_sh) = refs
        else:
            (a_hbm, src_hbm, dst_hbm, z2_hbm, out_hbm,
             src_v, dst_v, rows_v, acc_sh, sem) = refs
        c = lax.axis_index("c")
        s = lax.axis_index("s")

        base_r = pl.multiple_of(s * rows_pt, 8)
        pltpu.sync_copy(z2_hbm.at[pl.ds(base_r, rows_pt)],
                        acc_sh.at[pl.ds(base_r, rows_pt)])
        if tail:
            @pl.when(s == NS - 1)
            def _():
                pltpu.sync_copy(z2_hbm.at[pl.ds(NS * rows_pt, tail)],
                                acc_sh.at[pl.ds(NS * rows_pt, tail)])
        if compute_deg:
            @pl.when(s == 0)
            def _():
                pltpu.sync_copy(z1_hbm, deg_sh)
            for i in range(CH // 16):
                ones_v[pl.ds(i * 16, 16)] = jnp.ones((16,), F32)

        wid = c * NS + s
        pltpu.sync_copy(src_hbm.at[wid], src_v)
        pltpu.sync_copy(dst_hbm.at[wid], dst_v)
        plsc.subcore_barrier()

        def step(i, carry):
            cps = [pltpu.async_copy(a_hbm.at[src_v.at[i * G + j]],
                                    rows_v.at[j], sem)
                   for j in range(G)]
            for cp in cps:
                cp.wait()
            for j in range(G):
                pltpu.sync_copy(rows_v.at[j], acc_sh.at[dst_v.at[i * G + j]],
                                add=True)
                if compute_deg:
                    pltpu.sync_copy(ones_v, deg_sh.at[dst_v.at[i * G + j]],
                                    add=True)
            return carry

        lax.fori_loop(0, iters, step, 0)
        plsc.subcore_barrier()

        pltpu.sync_copy(acc_sh.at[pl.ds(base_r, rows_pt)],
                        out_hbm.at[c, pl.ds(base_r, rows_pt)])
        if tail:
            @pl.when(s == NS - 1)
            def _():
                pltpu.sync_copy(acc_sh.at[pl.ds(NS * rows_pt, tail)],
                                out_hbm.at[c, pl.ds(NS * rows_pt, tail)])
        if compute_deg:
            @pl.when(s == 0)
            def _():
                pltpu.sync_copy(deg_sh, deg_hbm.at[c])

    return k


def _dotd(a, b):
    return jnp.dot(a, b, preferred_element_type=F32)


def _dot_hi(a, b):
    return jnp.dot(a, b, preferred_element_type=F32,
                   precision=lax.Precision.HIGHEST)


def _bn(pre, g, b):
    m = jnp.mean(pre, axis=0, keepdims=True)
    d = pre - m
    v = jnp.mean(d * d, axis=0, keepdims=True)
    return (pre - m) / jnp.sqrt(v + 1e-5) * g + b


def _pre0_body(parta_ref, partb_ref, degt_ref, x_ref, wl_ref, wr_ref, ws_ref,
               bl_ref, pre_ref, sk_ref):
    deg = jnp.maximum(degt_ref[:, 0:1] + degt_ref[:, 1:2], 1.0)
    mean = jnp.concatenate(
        [(parta_ref[0] + parta_ref[1]) / deg,
         (partb_ref[0] + partb_ref[1]) / deg], axis=1)
    xv = x_ref[...]
    pre_ref[...] = _dotd(mean, wl_ref[...]) + bl_ref[...] + _dotd(xv, wr_ref[...])
    sk_ref[...] = _dotd(xv, ws_ref[...])


def _pre_body(part_ref, degt_ref, h_ref, wl_ref, wr_ref, bl_ref, pre_ref):
    s = part_ref[0] + part_ref[1]
    deg = degt_ref[:, 0:1] + degt_ref[:, 1:2]
    mean = s / jnp.maximum(deg, 1.0)
    pre_ref[...] = (_dotd(mean, wl_ref[...]) + bl_ref[...]
                    + _dotd(h_ref[...], wr_ref[...]))


def _bnres_relu_body(pre_ref, sk_ref, g_ref, b_ref, h_ref):
    h_ref[...] = jnp.maximum(_bn(pre_ref[...], g_ref[...], b_ref[...]),
                             0.0) + sk_ref[...]


def _bnres_body(pre_ref, sk_ref, g_ref, b_ref, h_ref):
    h_ref[...] = _bn(pre_ref[...], g_ref[...], b_ref[...]) + sk_ref[...]


def _make_pool_mlp(n, h, bsz, kk, cn=1000, sb=64):
    nchunk = n // cn
    nsb = bsz // sb
    assert n % cn == 0 and bsz % sb == 0

    def body(h3_ref, bt_ref, key_ref, w1_ref, b1_ref, w2_ref, b2_ref,
             out_ref, dense_scr, taken_scr, m_scr, sel_scr, acc_scr, act_scr):
        h3 = h3_ref[...]
        neg = jnp.float32(-jnp.inf)
        taken_scr[...] = jnp.zeros((nchunk, cn), F32)

        for c in range(nchunk):
            blk = bt_ref[c:c + 1, :] // sb
            for b in range(nsb):
                act_scr[c, b] = jnp.max(jnp.where(blk == b, 1, 0))

        iot_seg = lax.broadcasted_iota(jnp.int32, (sb, cn), 0)
        iot_col = lax.broadcasted_iota(jnp.int32, (sb, cn), 1)
        sel_scr[...] = jnp.full((bsz, 1), n + nchunk * cn, jnp.int32)

        def pass_k(k, carry):
            m_scr[...] = jnp.full((bsz, 1), neg, F32)
            acc_scr[...] = jnp.zeros((bsz, h), F32)
            for c in range(nchunk):
                for b in range(nsb):
                    @pl.when(act_scr[c, b] > 0)
                    def _():
                        selc = sel_scr[b * sb:(b + 1) * sb, :] - c * cn
                        oh = (iot_col == selc).astype(F32)
                        acc_scr[b * sb:(b + 1) * sb, :] += _dot_hi(
                            oh, h3[c * cn:(c + 1) * cn])
                        hit = jnp.max(oh, axis=0, keepdims=True)
                        taken = jnp.maximum(taken_scr[c:c + 1, :], hit)
                        taken_scr[c:c + 1, :] = taken
                        maskc = ((bt_ref[c:c + 1, :] - b * sb) == iot_seg) \
                            & (taken == 0.0)
                        mloc = jnp.max(
                            jnp.where(maskc, key_ref[c:c + 1, :], neg),
                            axis=1, keepdims=True)
                        m_scr[b * sb:(b + 1) * sb, :] = jnp.maximum(
                            m_scr[b * sb:(b + 1) * sb, :], mloc)

            @pl.when(k > 0)
            def _():
                dense_scr[:, pl.ds(k - 1, 1), :] = jnp.reshape(
                    acc_scr[...], (bsz, 1, h))

            @pl.when(k < kk)
            def _():
                sel_scr[...] = jnp.full((bsz, 1), n + nchunk * cn, jnp.int32)
                for c in range(nchunk):
                    for b in range(nsb):
                        @pl.when(act_scr[c, b] > 0)
                        def _():
                            free = taken_scr[c:c + 1, :] == 0.0
                            maskc = ((bt_ref[c:c + 1, :] - b * sb)
                                     == iot_seg) & free
                            eq = maskc & (key_ref[c:c + 1, :]
                                          == m_scr[b * sb:(b + 1) * sb, :])
                            sloc = jnp.min(jnp.where(eq, iot_col, n), axis=1,
                                           keepdims=True) + c * cn
                            sel_scr[b * sb:(b + 1) * sb, :] = jnp.minimum(
                                sel_scr[b * sb:(b + 1) * sb, :], sloc)
            return carry

        lax.fori_loop(0, kk + 1, pass_k, 0)
        pooled = jnp.reshape(dense_scr[...], (bsz, kk * h))
        hid = jnp.maximum(_dotd(pooled, w1_ref[...]) + b1_ref[...], 0.0)
        out_ref[...] = _dotd(hid, w2_ref[...]) + b2_ref[...]

    return body


def kernel(x, edge_index, batch, params):
    n, d = x.shape
    e = edge_index.shape[1]
    hdim = params["conv0_Wl"].shape[1]
    kk = params["mlp_W1"].shape[0] // hdim
    bsz = 256

    nw = NC * NS
    src2 = edge_index[0].reshape(nw, e // nw // CH, CH)
    dst2 = edge_index[1].reshape(nw, e // nw // CH, CH)
    zd = jnp.zeros((n, d), F32)
    zh = jnp.zeros((n, hdim), F32)
    z1 = jnp.zeros((n,), F32)
    cn = 1000
    bt2 = batch.reshape(n // cn, cn)

    def row(v):
        return v.reshape(1, -1)

    rb = 2000

    def wspec(*shape):
        return pl.BlockSpec(shape, lambda i: tuple(0 for _ in shape))

    dh = d // 2
    zdh = jnp.zeros((n, dh), F32)
    seg0 = _make_segsum(n, e, dh, True)
    part0a, degp = seg0(x[:, :dh], src2, dst2, zdh, z1)
    part0b = _make_segsum(n, e, dh, False)(x[:, dh:], src2, dst2, zdh)[0]
    degt = degp.T

    pre0, sk = pl.pallas_call(
        _pre0_body,
        grid=(n // rb,),
        in_specs=[
            pl.BlockSpec((NC, rb, dh), lambda i: (0, i, 0)),
            pl.BlockSpec((NC, rb, dh), lambda i: (0, i, 0)),
            pl.BlockSpec((rb, NC), lambda i: (i, 0)),
            pl.BlockSpec((rb, d), lambda i: (i, 0)),
            wspec(d, hdim), wspec(d, hdim), wspec(d, hdim), wspec(1, hdim),
        ],
        out_specs=[pl.BlockSpec((rb, hdim), lambda i: (i, 0))] * 2,
        out_shape=[jax.ShapeDtypeStruct((n, hdim), F32)] * 2,
    )(part0a, part0b, degt, x, params["conv0_Wl"], params["conv0_Wr"],
      params["skip_W"], row(params["conv0_bl"]))

    h1 = pl.pallas_call(
        _bnres_relu_body,
        out_shape=jax.ShapeDtypeStruct((n, hdim), F32),
    )(pre0, sk, row(params["bn0_g"]), row(params["bn0_b"]))

    seg = _make_segsum(n, e, hdim, False)

    def conv_layer(hv, l, relu):
        part = seg(hv, src2, dst2, zh)[0]
        pre = pl.pallas_call(
            _pre_body,
            grid=(n // rb,),
            in_specs=[
                pl.BlockSpec((NC, rb, hdim), lambda i: (0, i, 0)),
                pl.BlockSpec((rb, NC), lambda i: (i, 0)),
                pl.BlockSpec((rb, hdim), lambda i: (i, 0)),
                wspec(hdim, hdim), wspec(hdim, hdim), wspec(1, hdim),
            ],
            out_specs=pl.BlockSpec((rb, hdim), lambda i: (i, 0)),
            out_shape=jax.ShapeDtypeStruct((n, hdim), F32),
        )(part, degt, hv, params[f"conv{l}_Wl"], params[f"conv{l}_Wr"],
          row(params[f"conv{l}_bl"]))
        return pl.pallas_call(
            _bnres_relu_body if relu else _bnres_body,
            out_shape=jax.ShapeDtypeStruct((n, hdim), F32),
        )(pre, hv, row(params[f"bn{l}_g"]), row(params[f"bn{l}_b"]))

    h2 = conv_layer(h1, 1, True)
    h3 = conv_layer(h2, 2, False)

    key2 = h3[:, hdim - 1].reshape(n // cn, cn)
    sb = 64
    out = pl.pallas_call(
        _make_pool_mlp(n, hdim, bsz, kk, cn, sb),
        out_shape=jax.ShapeDtypeStruct((bsz, 1), F32),
        scratch_shapes=[pltpu.VMEM((bsz, kk, hdim), F32),
                        pltpu.VMEM((n // cn, cn), F32),
                        pltpu.VMEM((bsz, 1), F32),
                        pltpu.VMEM((bsz, 1), jnp.int32),
                        pltpu.VMEM((bsz, hdim), F32),
                        pltpu.SMEM((n // cn, bsz // sb), jnp.int32)],
    )(h3, bt2, key2, params["mlp_W1"], row(params["mlp_b1"]),
      params["mlp_W2"], row(params["mlp_b2"]))

    return out[:, 0]


# speedup vs baseline: 1.7495x; 1.1403x over previous
import functools

import jax
import jax.numpy as jnp
from jax import lax
from jax.experimental import pallas as pl
from jax.experimental.pallas import tpu as pltpu
from jax.experimental.pallas import tpu_sc as plsc

F32 = jnp.float32

NC = 2
NS = 16
CH = 80
G = 5


def _make_segsum(n, e, w, compute_deg):
    nw = NC * NS
    epw = e // nw
    rows_pt = (n // NS) // 8 * 8
    tail = n - NS * rows_pt
    iters = epw // (CH * G)
    assert epw % (CH * G) == 0 and tail % 8 == 0

    mesh = plsc.VectorSubcoreMesh(core_axis_name="c", subcore_axis_name="s")

    out_type = [jax.ShapeDtypeStruct((NC, n, w), F32)]
    if compute_deg:
        out_type.append(jax.ShapeDtypeStruct((NC, n), F32))

    scratch = [
        pltpu.VMEM((epw // CH, CH), jnp.int32),
        pltpu.VMEM((epw // CH, CH), jnp.int32),
        pltpu.VMEM((G, CH, w), F32),
        pltpu.VMEM_SHARED((n, w), F32),
        pltpu.SemaphoreType.DMA,
    ]
    if compute_deg:
        scratch += [
            pltpu.VMEM((CH,), F32),
            pltpu.VMEM_SHARED((n,), F32),
        ]

    @functools.partial(
        pl.kernel, mesh=mesh, out_type=out_type, scratch_types=scratch,
        name="sc_segsum_deg" if compute_deg else "sc_segsum",
        compiler_params=pltpu.CompilerParams(use_tc_tiling_on_sc=False),
    )
    def k(*refs):
        if compute_deg:
            (a_hbm, src_hbm, dst_hbm, z2_hbm, z1_hbm, out_hbm, deg_hbm,
             src_v, dst_v, rows_v, acc_sh, sem, ones_v, deg_sh) = refs
        else:
            (a_hbm, src_hbm, dst_hbm, z2_hbm, out_hbm,
             src_v, dst_v, rows_v, acc_sh, sem) = refs
        c = lax.axis_index("c")
        s = lax.axis_index("s")

        base_r = pl.multiple_of(s * rows_pt, 8)
        pltpu.sync_copy(z2_hbm.at[pl.ds(base_r, rows_pt)],
                        acc_sh.at[pl.ds(base_r, rows_pt)])
        if tail:
            @pl.when(s == NS - 1)
            def _():
                pltpu.sync_copy(z2_hbm.at[pl.ds(NS * rows_pt, tail)],
                                acc_sh.at[pl.ds(NS * rows_pt, tail)])
        if compute_deg:
            @pl.when(s == 0)
            def _():
                pltpu.sync_copy(z1_hbm, deg_sh)
            for i in range(CH // 16):
                ones_v[pl.ds(i * 16, 16)] = jnp.ones((16,), F32)

        wid = c * NS + s
        pltpu.sync_copy(src_hbm.at[wid], src_v)
        pltpu.sync_copy(dst_hbm.at[wid], dst_v)
        plsc.subcore_barrier()

        def step(i, carry):
            cps = [pltpu.async_copy(a_hbm.at[src_v.at[i * G + j]],
                                    rows_v.at[j], sem)
                   for j in range(G)]
            for cp in cps:
                cp.wait()
            for j in range(G):
                pltpu.sync_copy(rows_v.at[j], acc_sh.at[dst_v.at[i * G + j]],
                                add=True)
                if compute_deg:
                    pltpu.sync_copy(ones_v, deg_sh.at[dst_v.at[i * G + j]],
                                    add=True)
            return carry

        lax.fori_loop(0, iters, step, 0)
        plsc.subcore_barrier()

        pltpu.sync_copy(acc_sh.at[pl.ds(base_r, rows_pt)],
                        out_hbm.at[c, pl.ds(base_r, rows_pt)])
        if tail:
            @pl.when(s == NS - 1)
            def _():
                pltpu.sync_copy(acc_sh.at[pl.ds(NS * rows_pt, tail)],
                                out_hbm.at[c, pl.ds(NS * rows_pt, tail)])
        if compute_deg:
            @pl.when(s == 0)
            def _():
                pltpu.sync_copy(deg_sh, deg_hbm.at[c])

    return k


def _dotd(a, b):
    return jnp.dot(a, b, preferred_element_type=F32)


def _dot_hi(a, b):
    return jnp.dot(a, b, preferred_element_type=F32,
                   precision=lax.Precision.HIGHEST)


def _bn(pre, g, b):
    m = jnp.mean(pre, axis=0, keepdims=True)
    d = pre - m
    v = jnp.mean(d * d, axis=0, keepdims=True)
    return (pre - m) / jnp.sqrt(v + 1e-5) * g + b


def _pre0_body(parta_ref, partb_ref, degt_ref, x_ref, wl_ref, wr_ref, ws_ref,
               bl_ref, pre_ref, sk_ref):
    deg = jnp.maximum(degt_ref[:, 0:1] + degt_ref[:, 1:2], 1.0)
    mean = jnp.concatenate(
        [(parta_ref[0] + parta_ref[1]) / deg,
         (partb_ref[0] + partb_ref[1]) / deg], axis=1)
    xv = x_ref[...]
    pre_ref[...] = _dotd(mean, wl_ref[...]) + bl_ref[...] + _dotd(xv, wr_ref[...])
    sk_ref[...] = _dotd(xv, ws_ref[...])


def _pre_body(part_ref, degt_ref, h_ref, wl_ref, wr_ref, bl_ref, pre_ref):
    s = part_ref[0] + part_ref[1]
    deg = degt_ref[:, 0:1] + degt_ref[:, 1:2]
    mean = s / jnp.maximum(deg, 1.0)
    pre_ref[...] = (_dotd(mean, wl_ref[...]) + bl_ref[...]
                    + _dotd(h_ref[...], wr_ref[...]))


def _bnres_relu_body(pre_ref, sk_ref, g_ref, b_ref, h_ref):
    h_ref[...] = jnp.maximum(_bn(pre_ref[...], g_ref[...], b_ref[...]),
                             0.0) + sk_ref[...]


def _bnres_body(pre_ref, sk_ref, g_ref, b_ref, h_ref):
    h_ref[...] = _bn(pre_ref[...], g_ref[...], b_ref[...]) + sk_ref[...]


def _make_pool_mlp(n, h, bsz, kk, cn=1000, sb=64):
    nchunk = n // cn
    nsb = bsz // sb
    assert n % cn == 0 and bsz % sb == 0

    def body(h3_ref, bt_ref, key_ref, w1_ref, b1_ref, w2_ref, b2_ref,
             out_ref, dense_scr, taken_scr, m_scr, sel_scr, acc_scr, act_scr):
        h3 = h3_ref[...]
        h3hi = h3.astype(jnp.bfloat16).astype(F32)
        h3lo = h3 - h3hi
        neg = jnp.float32(-jnp.inf)
        taken_scr[...] = jnp.zeros((nchunk, cn), F32)

        for c in range(nchunk):
            blk = bt_ref[c:c + 1, :] // sb
            for b in range(nsb):
                act_scr[c, b] = jnp.max(jnp.where(blk == b, 1, 0))

        iot_seg = lax.broadcasted_iota(jnp.int32, (sb, cn), 0)
        iot_col = lax.broadcasted_iota(jnp.int32, (sb, cn), 1)
        sel_scr[...] = jnp.full((bsz, 1), n + nchunk * cn, jnp.int32)

        def pass_k(k, carry):
            m_scr[...] = jnp.full((bsz, 1), neg, F32)
            acc_scr[...] = jnp.zeros((bsz, h), F32)
            for c in range(nchunk):
                for b in range(nsb):
                    @pl.when(act_scr[c, b] > 0)
                    def _():
                        selc = sel_scr[b * sb:(b + 1) * sb, :] - c * cn
                        oh = (iot_col == selc).astype(F32)
                        acc_scr[b * sb:(b + 1) * sb, :] += (
                            _dotd(oh, h3hi[c * cn:(c + 1) * cn])
                            + _dotd(oh, h3lo[c * cn:(c + 1) * cn]))
                        hit = jnp.max(oh, axis=0, keepdims=True)
                        taken = jnp.maximum(taken_scr[c:c + 1, :], hit)
                        taken_scr[c:c + 1, :] = taken
                        maskc = ((bt_ref[c:c + 1, :] - b * sb) == iot_seg) \
                            & (taken == 0.0)
                        mloc = jnp.max(
                            jnp.where(maskc, key_ref[c:c + 1, :], neg),
                            axis=1, keepdims=True)
                        m_scr[b * sb:(b + 1) * sb, :] = jnp.maximum(
                            m_scr[b * sb:(b + 1) * sb, :], mloc)

            @pl.when(k > 0)
            def _():
                dense_scr[:, pl.ds(k - 1, 1), :] = jnp.reshape(
                    acc_scr[...], (bsz, 1, h))

            @pl.when(k < kk)
            def _():
                sel_scr[...] = jnp.full((bsz, 1), n + nchunk * cn, jnp.int32)
                for c in range(nchunk):
                    for b in range(nsb):
                        @pl.when(act_scr[c, b] > 0)
                        def _():
                            free = taken_scr[c:c + 1, :] == 0.0
                            maskc = ((bt_ref[c:c + 1, :] - b * sb)
                                     == iot_seg) & free
                            eq = maskc & (key_ref[c:c + 1, :]
                                          == m_scr[b * sb:(b + 1) * sb, :])
                            sloc = jnp.min(jnp.where(eq, iot_col, n), axis=1,
                                           keepdims=True) + c * cn
                            sel_scr[b * sb:(b + 1) * sb, :] = jnp.minimum(
                                sel_scr[b * sb:(b + 1) * sb, :], sloc)
            return carry

        lax.fori_loop(0, kk + 1, pass_k, 0)
        pooled = jnp.reshape(dense_scr[...], (bsz, kk * h))
        hid = jnp.maximum(_dotd(pooled, w1_ref[...]) + b1_ref[...], 0.0)
        out_ref[...] = _dotd(hid, w2_ref[...]) + b2_ref[...]

    return body


def kernel(x, edge_index, batch, params):
    n, d = x.shape
    e = edge_index.shape[1]
    hdim = params["conv0_Wl"].shape[1]
    kk = params["mlp_W1"].shape[0] // hdim
    bsz = 256

    nw = NC * NS
    src2 = edge_index[0].reshape(nw, e // nw // CH, CH)
    dst2 = edge_index[1].reshape(nw, e // nw // CH, CH)
    zd = jnp.zeros((n, d), F32)
    zh = jnp.zeros((n, hdim), F32)
    z1 = jnp.zeros((n,), F32)
    cn = 1000
    bt2 = batch.reshape(n // cn, cn)

    def row(v):
        return v.reshape(1, -1)

    rb = 2000

    def wspec(*shape):
        return pl.BlockSpec(shape, lambda i: tuple(0 for _ in shape))

    dh = d // 2
    zdh = jnp.zeros((n, dh), F32)
    seg0 = _make_segsum(n, e, dh, True)
    part0a, degp = seg0(x[:, :dh], src2, dst2, zdh, z1)
    part0b = _make_segsum(n, e, dh, False)(x[:, dh:], src2, dst2, zdh)[0]
    degt = degp.T

    pre0, sk = pl.pallas_call(
        _pre0_body,
        grid=(n // rb,),
        in_specs=[
            pl.BlockSpec((NC, rb, dh), lambda i: (0, i, 0)),
            pl.BlockSpec((NC, rb, dh), lambda i: (0, i, 0)),
            pl.BlockSpec((rb, NC), lambda i: (i, 0)),
            pl.BlockSpec((rb, d), lambda i: (i, 0)),
            wspec(d, hdim), wspec(d, hdim), wspec(d, hdim), wspec(1, hdim),
        ],
        out_specs=[pl.BlockSpec((rb, hdim), lambda i: (i, 0))] * 2,
        out_shape=[jax.ShapeDtypeStruct((n, hdim), F32)] * 2,
    )(part0a, part0b, degt, x, params["conv0_Wl"], params["conv0_Wr"],
      params["skip_W"], row(params["conv0_bl"]))

    h1 = pl.pallas_call(
        _bnres_relu_body,
        out_shape=jax.ShapeDtypeStruct((n, hdim), F32),
    )(pre0, sk, row(params["bn0_g"]), row(params["bn0_b"]))

    seg = _make_segsum(n, e, hdim, False)

    def conv_layer(hv, l, relu):
        part = seg(hv, src2, dst2, zh)[0]
        pre = pl.pallas_call(
            _pre_body,
            grid=(n // rb,),
            in_specs=[
                pl.BlockSpec((NC, rb, hdim), lambda i: (0, i, 0)),
                pl.BlockSpec((rb, NC), lambda i: (i, 0)),
                pl.BlockSpec((rb, hdim), lambda i: (i, 0)),
                wspec(hdim, hdim), wspec(hdim, hdim), wspec(1, hdim),
            ],
            out_specs=pl.BlockSpec((rb, hdim), lambda i: (i, 0)),
            out_shape=jax.ShapeDtypeStruct((n, hdim), F32),
        )(part, degt, hv, params[f"conv{l}_Wl"], params[f"conv{l}_Wr"],
          row(params[f"conv{l}_bl"]))
        return pl.pallas_call(
            _bnres_relu_body if relu else _bnres_body,
            out_shape=jax.ShapeDtypeStruct((n, hdim), F32),
        )(pre, hv, row(params[f"bn{l}_g"]), row(params[f"bn{l}_b"]))

    h2 = conv_layer(h1, 1, True)
    h3 = conv_layer(h2, 2, False)

    key2 = h3[:, hdim - 1].reshape(n // cn, cn)
    sb = 64
    out = pl.pallas_call(
        _make_pool_mlp(n, hdim, bsz, kk, cn, sb),
        out_shape=jax.ShapeDtypeStruct((bsz, 1), F32),
        scratch_shapes=[pltpu.VMEM((bsz, kk, hdim), F32),
                        pltpu.VMEM((n // cn, cn), F32),
                        pltpu.VMEM((bsz, 1), F32),
                        pltpu.VMEM((bsz, 1), jnp.int32),
                        pltpu.VMEM((bsz, hdim), F32),
                        pltpu.SMEM((n // cn, bsz // sb), jnp.int32)],
    )(h3, bt2, key2, params["mlp_W1"], row(params["mlp_b1"]),
      params["mlp_W2"], row(params["mlp_b2"]))

    return out[:, 0]
